# baseline (speedup 1.0000x reference)
import functools

import jax
import jax.numpy as jnp
from jax import lax
from jax.experimental import pallas as pl
from jax.experimental.pallas import tpu as pltpu
from jax.experimental.pallas import tpu_sc as plsc

N_SPECIES = 1000000
EMBED_DIM = 32
BATCH = 16384
OUT_DIM = EMBED_DIM + 3
OUT_PAD = 128
SINK = BATCH * OUT_DIM

NC = 2
NS = 16
NW = NC * NS

CHUNK = 1024
NCH_TOTAL = 977
BASE_CH = NCH_TOTAL // NW
EXTRA = NCH_TOTAL - BASE_CH * NW
ALIGN_BASE = 999040

HITCAP = 752
ROWSTRIDE = 48
CHCAP = 80
NVH = HITCAP // 16


def _body(idx_hbm, g_hbm, s_hbm, c_hbm, tabT_hbm, out_hbm,
          idx_v, chunk_v, rows_v, hit_r, hit_b, crc, cbc,
          sb_raw, sb_cl, gt_v, st_v, ct_v, semc, semo, semi):
    c = lax.axis_index("c")
    s = lax.axis_index("s")
    wid = s * NC + c
    lanes = lax.iota(jnp.int32, 16)
    i16384 = jnp.full((16,), BATCH, jnp.int32)

    nch = BASE_CH + jnp.where(wid < EXTRA, 1, 0)
    ch0 = BASE_CH * wid + jnp.minimum(wid, EXTRA)
    lane_lo = ch0 * CHUNK
    lane_hi = jnp.minimum((ch0 + nch) * CHUNK, N_SPECIES)

    def chunk_dma(ci, buf):
        lo = (ch0 + ci) * CHUNK
        base = jnp.minimum(lo, ALIGN_BASE)
        base = pl.multiple_of(base, 128)
        return pltpu.async_copy(
            tabT_hbm.at[:, pl.ds(base, CHUNK)], chunk_v.at[buf], semc)

    chunk_dma(0, 0)
    chunk_dma(jnp.minimum(1, nch - 1), 1)

    pltpu.sync_copy(idx_hbm, idx_v)

    neg1 = jnp.full((16,), -1, jnp.int32)
    for k in range(NVH):
        plsc.store_scatter(hit_r, [16 * k + lanes], neg1)
        plsc.store_scatter(hit_b, [16 * k + lanes], i16384)
        plsc.store_scatter(sb_raw, [16 * k + lanes], i16384)
        plsc.store_scatter(sb_cl, [16 * k + lanes], i16384 - 1)

    def scan_step(i, cnt):
        v = idx_v[pl.ds(i * 16, 16)]
        m = jnp.logical_and(v >= lane_lo, v < lane_hi)
        m32 = m.astype(jnp.int32)
        pos = cnt + jnp.cumsum(m32) - m32
        plsc.store_scatter(hit_r, [pos], v, mask=m)
        plsc.store_scatter(hit_b, [pos], i * 16 + lanes, mask=m)
        return cnt + jnp.sum(m32)

    nhit = lax.fori_loop(0, BATCH // 16, scan_step, jnp.int32(0),
                         unroll=False)

    def per_chunk(ci, gsc):
        lo = (ch0 + ci) * CHUNK
        base = jnp.minimum(lo, ALIGN_BASE)
        hi = jnp.minimum(lo + CHUNK, N_SPECIES)
        buf = lax.rem(ci, 2)
        pltpu.make_async_copy(tabT_hbm.at[:, pl.ds(0, CHUNK)],
                              chunk_v.at[buf], semc).wait()

        for k in range(CHCAP // 16):
            plsc.store_scatter(crc, [16 * k + lanes], base + (lanes * 0))
            plsc.store_scatter(cbc, [16 * k + lanes], i16384)

        def sel_step(t, ck):
            hv = hit_r[pl.ds(t * 16, 16)]
            bv = hit_b[pl.ds(t * 16, 16)]
            m = jnp.logical_and(hv >= lo, hv < hi)
            m32 = m.astype(jnp.int32)
            pos = ck + jnp.cumsum(m32) - m32
            plsc.store_scatter(crc, [pos], hv, mask=m)
            plsc.store_scatter(cbc, [pos], bv, mask=m)
            return ck + jnp.sum(m32)

        ck = lax.fori_loop(0, NVH, sel_step, jnp.int32(0), unroll=False)

        zeros16 = jnp.full((16,), 0, jnp.int32)

        def ext_step(t, _):
            rv = crc[pl.ds(t * 16, 16)]
            bv = cbc[pl.ds(t * 16, 16)]
            sl0 = gsc + t * 16
            plsc.store_scatter(sb_raw, [sl0 + lanes], bv)
            plsc.store_scatter(sb_cl, [sl0 + lanes],
                              jnp.minimum(bv, BATCH - 1))
            for j in range(16):
                lm = rv[j] - base
                lmv = zeros16 + lm
                bufv = zeros16 + buf
                lo16 = plsc.load_gather(chunk_v, [bufv, lanes, lmv])
                hi16 = plsc.load_gather(chunk_v, [bufv, lanes + 16, lmv])
                slot = sl0 + j
                plsc.store_scatter(rows_v, [slot * ROWSTRIDE + lanes], lo16)
                plsc.store_scatter(rows_v, [slot * ROWSTRIDE + 16 + lanes],
                                   hi16)
            return 0

        nv = lax.shift_right_logical(ck + 15, 4)
        lax.fori_loop(0, nv, ext_step, 0, unroll=False)

        chunk_dma(jnp.minimum(ci + 2, nch - 1), buf)
        return gsc + ck

    gsc = lax.fori_loop(0, nch, per_chunk, jnp.int32(0), unroll=False)
    pltpu.make_async_copy(tabT_hbm.at[:, pl.ds(0, CHUNK)],
                          chunk_v.at[0], semc).wait()
    pltpu.make_async_copy(tabT_hbm.at[:, pl.ds(0, CHUNK)],
                          chunk_v.at[1], semc).wait()

    gcps = []
    for k in range(HITCAP // 128 + 1):
        o = min(128 * k, HITCAP - 128)
        gcps.append(pltpu.async_copy(g_hbm.at[sb_cl.at[pl.ds(o, 128)]],
                                     gt_v.at[pl.ds(o, 128)], semi))
        gcps.append(pltpu.async_copy(s_hbm.at[sb_cl.at[pl.ds(o, 128)]],
                                     st_v.at[pl.ds(o, 128)], semi))
        gcps.append(pltpu.async_copy(c_hbm.at[sb_cl.at[pl.ds(o, 128)]],
                                     ct_v.at[pl.ds(o, 128)], semi))
    for cp in gcps:
        cp.wait()
    for k in range(NVH):
        dst = (16 * k + lanes) * ROWSTRIDE + EMBED_DIM
        plsc.store_scatter(rows_v, [dst], gt_v[pl.ds(16 * k, 16)])
        plsc.store_scatter(rows_v, [dst + 1], st_v[pl.ds(16 * k, 16)])
        plsc.store_scatter(rows_v, [dst + 2], ct_v[pl.ds(16 * k, 16)])

    def out_step(t, _):
        bv = sb_raw[pl.ds(t * 16, 16)]
        cps = []
        for j in range(16):
            b35 = bv[j] * OUT_DIM
            slot = t * 16 + j
            d0 = b35 + lanes
            d1 = b35 + 16 + lanes
            d2 = jnp.where(lanes < OUT_DIM - 24, b35 + 24 + lanes,
                           SINK + lanes)
            src = rows_v.at[pl.ds(slot * ROWSTRIDE, 16)]
            cps.append(pltpu.async_copy(src, out_hbm.at[d0], semo))
            src = rows_v.at[pl.ds(slot * ROWSTRIDE + 16, 16)]
            cps.append(pltpu.async_copy(src, out_hbm.at[d1], semo))
            src = rows_v.at[pl.ds(slot * ROWSTRIDE + 24, 16)]
            cps.append(pltpu.async_copy(src, out_hbm.at[d2], semo))
        for cp in cps:
            cp.wait()
        return 0

    nf = lax.shift_right_logical(gsc + 31, 4)
    lax.fori_loop(0, nf, out_step, 0, unroll=False)


@functools.partial(jax.jit, static_argnums=())
def kernel(species_idx, group_idx, sin_date, cos_date, species_embedding):
    mesh = plsc.VectorSubcoreMesh(core_axis_name="c", subcore_axis_name="s")
    run = pl.kernel(
        _body,
        mesh=mesh,
        compiler_params=pltpu.CompilerParams(needs_layout_passes=False),
        out_type=jax.ShapeDtypeStruct((BATCH * OUT_DIM + OUT_PAD,),
                                      jnp.float32),
        scratch_types=[
            pltpu.VMEM((BATCH,), jnp.int32),
            pltpu.VMEM((2, EMBED_DIM, CHUNK), jnp.float32),
            pltpu.VMEM((HITCAP * ROWSTRIDE,), jnp.float32),
            pltpu.VMEM((HITCAP,), jnp.int32),
            pltpu.VMEM((HITCAP,), jnp.int32),
            pltpu.VMEM((CHCAP,), jnp.int32),
            pltpu.VMEM((CHCAP,), jnp.int32),
            pltpu.VMEM((HITCAP,), jnp.int32),
            pltpu.VMEM((HITCAP,), jnp.int32),
            pltpu.VMEM((HITCAP,), jnp.float32),
            pltpu.VMEM((HITCAP,), jnp.float32),
            pltpu.VMEM((HITCAP,), jnp.float32),
            pltpu.SemaphoreType.DMA,
            pltpu.SemaphoreType.DMA,
            pltpu.SemaphoreType.DMA,
        ],
    )
    flat = run(species_idx.astype(jnp.int32), group_idx, sin_date, cos_date,
               species_embedding.T)
    return flat[:BATCH * OUT_DIM].reshape(BATCH, OUT_DIM)


# speedup vs baseline: 1.0001x; 1.0001x over previous
import functools

import jax
import jax.numpy as jnp
from jax import lax
from jax.experimental import pallas as pl
from jax.experimental.pallas import tpu as pltpu
from jax.experimental.pallas import tpu_sc as plsc

N_SPECIES = 1000000
EMBED_DIM = 32
BATCH = 16384
OUT_DIM = EMBED_DIM + 3
OUT_PAD = 128
SINK = BATCH * OUT_DIM

NC = 2
NS = 16
NW = NC * NS

CHUNK = 1024
NCH_TOTAL = 977
BASE_CH = NCH_TOTAL // NW
EXTRA = NCH_TOTAL - BASE_CH * NW
ALIGN_BASE = 999040

HITCAP = 752
ROWSTRIDE = 48
CHCAP = 80
NVH = HITCAP // 16


def _body(idx_hbm, g_hbm, s_hbm, c_hbm, tabT_hbm, out_hbm,
          idx_v, chunk_v, rows_v, hit_r, hit_b, crc, cbc,
          sb_raw, sb_cl, gt_v, st_v, ct_v, semc, semo, semi):
    c = lax.axis_index("c")
    s = lax.axis_index("s")
    wid = s * NC + c
    lanes = lax.iota(jnp.int32, 16)
    i16384 = jnp.full((16,), BATCH, jnp.int32)

    nch = BASE_CH + jnp.where(wid < EXTRA, 1, 0)
    ch0 = BASE_CH * wid + jnp.minimum(wid, EXTRA)
    lane_lo = ch0 * CHUNK
    lane_hi = jnp.minimum((ch0 + nch) * CHUNK, N_SPECIES)

    def chunk_dma(ci, buf):
        lo = (ch0 + ci) * CHUNK
        base = jnp.minimum(lo, ALIGN_BASE)
        base = pl.multiple_of(base, 128)
        return pltpu.async_copy(
            tabT_hbm.at[:, pl.ds(base, CHUNK)], chunk_v.at[buf], semc)

    chunk_dma(0, 0)
    chunk_dma(jnp.minimum(1, nch - 1), 1)

    pltpu.sync_copy(idx_hbm, idx_v)

    neg1 = jnp.full((16,), -1, jnp.int32)
    for k in range(NVH):
        plsc.store_scatter(hit_r, [16 * k + lanes], neg1)
        plsc.store_scatter(hit_b, [16 * k + lanes], i16384)
        plsc.store_scatter(sb_raw, [16 * k + lanes], i16384)
        plsc.store_scatter(sb_cl, [16 * k + lanes], i16384 - 1)

    def scan_step(i, cnt):
        v = idx_v[pl.ds(i * 16, 16)]
        m = jnp.logical_and(v >= lane_lo, v < lane_hi)
        m32 = m.astype(jnp.int32)
        cs = jnp.cumsum(m32)
        pos = cnt + cs - m32
        plsc.store_scatter(hit_r, [pos], v, mask=m)
        plsc.store_scatter(hit_b, [pos], i * 16 + lanes, mask=m)
        return cnt + cs[15]

    nhit = lax.fori_loop(0, BATCH // 16, scan_step, jnp.int32(0),
                         unroll=False)

    def per_chunk(ci, gsc):
        lo = (ch0 + ci) * CHUNK
        base = jnp.minimum(lo, ALIGN_BASE)
        hi = jnp.minimum(lo + CHUNK, N_SPECIES)
        buf = lax.rem(ci, 2)
        pltpu.make_async_copy(tabT_hbm.at[:, pl.ds(0, CHUNK)],
                              chunk_v.at[buf], semc).wait()

        for k in range(CHCAP // 16):
            plsc.store_scatter(crc, [16 * k + lanes], base + (lanes * 0))
            plsc.store_scatter(cbc, [16 * k + lanes], i16384)

        def sel_step(t, ck):
            hv = hit_r[pl.ds(t * 16, 16)]
            m = jnp.logical_and(hv >= lo, hv < hi)

            def hit_branch(ck):
                bv = hit_b[pl.ds(t * 16, 16)]
                m32 = m.astype(jnp.int32)
                cs = jnp.cumsum(m32)
                pos = ck + cs - m32
                plsc.store_scatter(crc, [pos], hv, mask=m)
                plsc.store_scatter(cbc, [pos], bv, mask=m)
                return ck + cs[15]

            return lax.cond(jnp.any(m), hit_branch, lambda ck: ck, ck)

        nvsel = lax.shift_right_logical(nhit + 15, 4)
        ck = lax.fori_loop(0, nvsel, sel_step, jnp.int32(0), unroll=False)

        zeros16 = jnp.full((16,), 0, jnp.int32)

        def ext_step(t, _):
            rv = crc[pl.ds(t * 16, 16)]
            bv = cbc[pl.ds(t * 16, 16)]
            sl0 = gsc + t * 16
            plsc.store_scatter(sb_raw, [sl0 + lanes], bv)
            plsc.store_scatter(sb_cl, [sl0 + lanes],
                              jnp.minimum(bv, BATCH - 1))
            for j in range(16):
                lm = rv[j] - base
                lmv = zeros16 + lm
                bufv = zeros16 + buf
                lo16 = plsc.load_gather(chunk_v, [bufv, lanes, lmv])
                hi16 = plsc.load_gather(chunk_v, [bufv, lanes + 16, lmv])
                slot = sl0 + j
                plsc.store_scatter(rows_v, [slot * ROWSTRIDE + lanes], lo16)
                plsc.store_scatter(rows_v, [slot * ROWSTRIDE + 16 + lanes],
                                   hi16)
            return 0

        nv = lax.shift_right_logical(ck + 15, 4)
        lax.fori_loop(0, nv, ext_step, 0, unroll=False)

        chunk_dma(jnp.minimum(ci + 2, nch - 1), buf)
        return gsc + ck

    gsc = lax.fori_loop(0, nch, per_chunk, jnp.int32(0), unroll=False)
    pltpu.make_async_copy(tabT_hbm.at[:, pl.ds(0, CHUNK)],
                          chunk_v.at[0], semc).wait()
    pltpu.make_async_copy(tabT_hbm.at[:, pl.ds(0, CHUNK)],
                          chunk_v.at[1], semc).wait()

    gcps = []
    for k in range(HITCAP // 128 + 1):
        o = min(128 * k, HITCAP - 128)
        gcps.append(pltpu.async_copy(g_hbm.at[sb_cl.at[pl.ds(o, 128)]],
                                     gt_v.at[pl.ds(o, 128)], semi))
        gcps.append(pltpu.async_copy(s_hbm.at[sb_cl.at[pl.ds(o, 128)]],
                                     st_v.at[pl.ds(o, 128)], semi))
        gcps.append(pltpu.async_copy(c_hbm.at[sb_cl.at[pl.ds(o, 128)]],
                                     ct_v.at[pl.ds(o, 128)], semi))
    for cp in gcps:
        cp.wait()
    for k in range(NVH):
        dst = (16 * k + lanes) * ROWSTRIDE + EMBED_DIM
        plsc.store_scatter(rows_v, [dst], gt_v[pl.ds(16 * k, 16)])
        plsc.store_scatter(rows_v, [dst + 1], st_v[pl.ds(16 * k, 16)])
        plsc.store_scatter(rows_v, [dst + 2], ct_v[pl.ds(16 * k, 16)])

    def out_step(t, _):
        bv = sb_raw[pl.ds(t * 16, 16)]
        for j in range(16):
            b35 = bv[j] * OUT_DIM
            slot = t * 16 + j
            d0 = b35 + lanes
            d1 = b35 + 16 + lanes
            d2 = jnp.where(lanes < OUT_DIM - 24, b35 + 24 + lanes,
                           SINK + lanes)
            src = rows_v.at[pl.ds(slot * ROWSTRIDE, 16)]
            pltpu.async_copy(src, out_hbm.at[d0], semo)
            src = rows_v.at[pl.ds(slot * ROWSTRIDE + 16, 16)]
            pltpu.async_copy(src, out_hbm.at[d1], semo)
            src = rows_v.at[pl.ds(slot * ROWSTRIDE + 24, 16)]
            pltpu.async_copy(src, out_hbm.at[d2], semo)
        return 0

    def drain_step(t, _):
        for _j in range(48):
            pltpu.make_async_copy(g_hbm.at[pl.ds(0, 16)],
                                  rows_v.at[pl.ds(0, 16)], semo).wait()
        return 0

    nf = lax.shift_right_logical(gsc + 31, 4)
    lax.fori_loop(0, nf, out_step, 0, unroll=False)
    lax.fori_loop(0, nf, drain_step, 0, unroll=False)


@functools.partial(jax.jit, static_argnums=())
def kernel(species_idx, group_idx, sin_date, cos_date, species_embedding):
    mesh = plsc.VectorSubcoreMesh(core_axis_name="c", subcore_axis_name="s")
    run = pl.kernel(
        _body,
        mesh=mesh,
        compiler_params=pltpu.CompilerParams(needs_layout_passes=False),
        out_type=jax.ShapeDtypeStruct((BATCH * OUT_DIM + OUT_PAD,),
                                      jnp.float32),
        scratch_types=[
            pltpu.VMEM((BATCH,), jnp.int32),
            pltpu.VMEM((2, EMBED_DIM, CHUNK), jnp.float32),
            pltpu.VMEM((HITCAP * ROWSTRIDE,), jnp.float32),
            pltpu.VMEM((HITCAP,), jnp.int32),
            pltpu.VMEM((HITCAP,), jnp.int32),
            pltpu.VMEM((CHCAP,), jnp.int32),
            pltpu.VMEM((CHCAP,), jnp.int32),
            pltpu.VMEM((HITCAP,), jnp.int32),
            pltpu.VMEM((HITCAP,), jnp.int32),
            pltpu.VMEM((HITCAP,), jnp.float32),
            pltpu.VMEM((HITCAP,), jnp.float32),
            pltpu.VMEM((HITCAP,), jnp.float32),
            pltpu.SemaphoreType.DMA,
            pltpu.SemaphoreType.DMA,
            pltpu.SemaphoreType.DMA,
        ],
    )
    flat = run(species_idx.astype(jnp.int32), group_idx, sin_date, cos_date,
               species_embedding.T)
    return flat[:BATCH * OUT_DIM].reshape(BATCH, OUT_DIM)


# speedup vs baseline: 137.5657x; 137.5456x over previous
import functools

import jax
import jax.numpy as jnp
from jax import lax
from jax.experimental import pallas as pl
from jax.experimental.pallas import tpu as pltpu
from jax.experimental.pallas import tpu_sc as plsc

N_SPECIES = 1000000
EMBED_DIM = 32
BATCH = 16384
OUT_DIM = EMBED_DIM + 3

NC = 2
NS = 16
NW = NC * NS
BPW = BATCH // NW
GRP = 8
N_GRP = BPW // GRP


def _body(idx_hbm, g_hbm, s_hbm, c_hbm, tabT_hbm, out_hbm,
          idx_v, blocks_v, g_v, s_v, c_v, out_v, sem):
    c = lax.axis_index("c")
    s = lax.axis_index("s")
    wid = s * NC + c
    base = wid * BPW

    pltpu.sync_copy(idx_hbm.at[pl.ds(base, BPW)], idx_v.at[pl.ds(0, BPW)])
    pltpu.sync_copy(g_hbm.at[pl.ds(base, BPW)], g_v)
    pltpu.sync_copy(s_hbm.at[pl.ds(base, BPW)], s_v)
    pltpu.sync_copy(c_hbm.at[pl.ds(base, BPW)], c_v)

    lanes = lax.iota(jnp.int32, 16)

    def fire(gv, slot0):
        for j in range(GRP):
            r = gv[j]
            blk = lax.shift_right_logical(r, 7)
            off = pl.multiple_of(blk * 128, 128)
            pltpu.async_copy(tabT_hbm.at[:, pl.ds(off, 128)],
                             blocks_v.at[slot0 + j], sem)

    fire(idx_v[pl.ds(0, 16)], 0)

    def group(g, _):
        gn = jnp.minimum(g + 1, N_GRP - 1)
        fire(idx_v[pl.ds(gn * GRP, 16)], GRP * lax.rem(g + 1, 2))
        for _j in range(GRP):
            pltpu.make_async_copy(tabT_hbm.at[:, pl.ds(0, 128)],
                                  blocks_v.at[0], sem).wait()
        v16 = idx_v[pl.ds(g * GRP, 16)]
        slot0 = GRP * lax.rem(g, 2)
        for j in range(GRP):
            r = v16[j]
            lm = lax.bitwise_and(r, jnp.int32(127))
            jv = jnp.full((16,), 0, jnp.int32) + (slot0 + j)
            lmv = jnp.full((16,), 0, jnp.int32) + lm
            lo = plsc.load_gather(blocks_v, [jv, lanes, lmv])
            hi = plsc.load_gather(blocks_v, [jv, lanes + 16, lmv])
            dst = (g * GRP + j) * OUT_DIM + lanes
            plsc.store_scatter(out_v, [dst], lo)
            plsc.store_scatter(out_v, [dst + 16], hi)
        return 0

    lax.fori_loop(0, N_GRP, group, 0, unroll=False)
    for _j in range(GRP):
        pltpu.make_async_copy(tabT_hbm.at[:, pl.ds(0, 128)],
                              blocks_v.at[0], sem).wait()

    for gblk in range(BPW // 16):
        dst = (16 * gblk + lanes) * OUT_DIM + EMBED_DIM
        plsc.store_scatter(out_v, [dst], g_v[pl.ds(16 * gblk, 16)])
        plsc.store_scatter(out_v, [dst + 1], s_v[pl.ds(16 * gblk, 16)])
        plsc.store_scatter(out_v, [dst + 2], c_v[pl.ds(16 * gblk, 16)])

    pltpu.sync_copy(out_v, out_hbm.at[pl.ds(base * OUT_DIM, BPW * OUT_DIM)])


@functools.partial(jax.jit, static_argnums=())
def kernel(species_idx, group_idx, sin_date, cos_date, species_embedding):
    mesh = plsc.VectorSubcoreMesh(core_axis_name="c", subcore_axis_name="s")
    run = pl.kernel(
        _body,
        mesh=mesh,
        compiler_params=pltpu.CompilerParams(needs_layout_passes=False),
        out_type=jax.ShapeDtypeStruct((BATCH * OUT_DIM,), jnp.float32),
        scratch_types=[
            pltpu.VMEM((BPW + 16,), jnp.int32),
            pltpu.VMEM((2 * GRP, EMBED_DIM, 128), jnp.float32),
            pltpu.VMEM((BPW,), jnp.float32),
            pltpu.VMEM((BPW,), jnp.float32),
            pltpu.VMEM((BPW,), jnp.float32),
            pltpu.VMEM((BPW * OUT_DIM,), jnp.float32),
            pltpu.SemaphoreType.DMA,
        ],
    )
    flat = run(species_idx.astype(jnp.int32), group_idx, sin_date, cos_date,
               species_embedding.T)
    return flat.reshape(BATCH, OUT_DIM)
